# baseline (speedup 1.0000x reference)
import jax
import jax.numpy as jnp
from jax import lax
from jax.experimental import pallas as pl
from jax.experimental.pallas import tpu as pltpu
from jax.experimental.pallas import tpu_sc as plsc

NUM_FIELDS = 26
VOCAB_PER_FIELD = 100000
NUM_CLASSES = 8
BATCH = 16384

L = 16
NW = 32
ROWS_PER_W = BATCH // NW
CHUNK = 128
N_CHUNK = ROWS_PER_W // CHUNK


def _sc_body(idx_hbm, table_hbm, out_hbm, idx_v, gath_v, out_v, sem):
    cid = lax.axis_index("c")
    sid = lax.axis_index("s")
    wid = sid * 2 + cid
    base_b = wid * ROWS_PER_W

    pltpu.sync_copy(idx_hbm.at[:, pl.ds(base_b, ROWS_PER_W)], idx_v)

    def _off_body(f, _):
        off = f * VOCAB_PER_FIELD
        for v in range(ROWS_PER_W // L):
            sl = pl.ds(v * L, L)
            idx_v[f, sl] = idx_v[f, sl] + off
        return 0

    lax.fori_loop(0, NUM_FIELDS, _off_body, 0)

    for chunk in range(N_CHUNK):
        cbase = chunk * CHUNK

        def _fire(f):
            ids = idx_v.at[f, pl.ds(cbase, CHUNK)]
            for c in range(NUM_CLASSES):
                pltpu.make_async_copy(
                    table_hbm.at[c].at[ids], gath_v.at[f, c], sem).start()

        def _drain(f):
            ids = idx_v.at[f, pl.ds(cbase, CHUNK)]
            for c in range(NUM_CLASSES):
                pltpu.make_async_copy(
                    table_hbm.at[c].at[ids], gath_v.at[f, c], sem).wait()

        def _pipe(f, _):
            _fire(f)

            @pl.when(f > 0)
            def _():
                _drain(f - 1)

            return 0

        lax.fori_loop(0, NUM_FIELDS, _pipe, 0)
        _drain(NUM_FIELDS - 1)

        for c in range(NUM_CLASSES):
            for v in range(CHUNK // L):
                out_v[c, pl.ds(cbase + v * L, L)] = jnp.zeros((L,), jnp.float32)

        def _reduce(f, _):
            for c in range(NUM_CLASSES):
                for v in range(CHUNK // L):
                    sl = pl.ds(v * L, L)
                    osl = pl.ds(cbase + v * L, L)
                    out_v[c, osl] = out_v[c, osl] + gath_v[f, c, sl]
            return 0

        lax.fori_loop(0, NUM_FIELDS, _reduce, 0)

    pltpu.sync_copy(out_v, out_hbm.at[:, pl.ds(base_b, ROWS_PER_W)])


@jax.jit
def _run(idx_t, table_t):
    mesh = plsc.VectorSubcoreMesh(core_axis_name="c", subcore_axis_name="s")
    call = pl.kernel(
        _sc_body,
        mesh=mesh,
        compiler_params=pltpu.CompilerParams(use_tc_tiling_on_sc=False),
        out_type=jax.ShapeDtypeStruct((NUM_CLASSES, BATCH), jnp.float32),
        scratch_types=[
            pltpu.VMEM((NUM_FIELDS, ROWS_PER_W), jnp.int32),
            pltpu.VMEM((NUM_FIELDS, NUM_CLASSES, CHUNK), jnp.float32),
            pltpu.VMEM((NUM_CLASSES, ROWS_PER_W), jnp.float32),
            pltpu.SemaphoreType.DMA,
        ],
    )
    return call(idx_t, table_t)


def kernel(indices, table):
    idx_t = indices.astype(jnp.int32).T
    out_t = _run(idx_t, table.T)
    return out_t.T


# speedup vs baseline: 1.5116x; 1.5116x over previous
import jax
import jax.numpy as jnp
from jax import lax
from jax.experimental import pallas as pl
from jax.experimental.pallas import tpu as pltpu
from jax.experimental.pallas import tpu_sc as plsc

NUM_FIELDS = 26
VOCAB_PER_FIELD = 100000
NUM_CLASSES = 8
BATCH = 16384

L = 16
NW = 32
ROWS_PER_OP = 128
ROWS_PER_W = BATCH // NW
BLK_PER_W = ROWS_PER_W // ROWS_PER_OP
OPS_PER_W = NUM_FIELDS * BLK_PER_W


def _sc_body(idx_hbm, table_hbm, out_hbm, idx_v, rows_v, ident_v, tmp_v,
             acc_sh, sem0, sem1):
    cid = lax.axis_index("c")
    sid = lax.axis_index("s")
    wid = sid * 2 + cid

    pltpu.sync_copy(idx_hbm.at[:, pl.ds(wid * ROWS_PER_W, ROWS_PER_W)], idx_v)

    lanes = lax.iota(jnp.int32, L)

    for j in range(BLK_PER_W):
        for v in range(ROWS_PER_OP // L):
            ident_v[j, pl.ds(v * L, L)] = lanes + (
                sid * ROWS_PER_W + j * ROWS_PER_OP + v * L)

    def _off_body(f, _):
        off = f * VOCAB_PER_FIELD
        for v in range(ROWS_PER_W // L):
            sl = pl.ds(v * L, L)
            idx_v[f, sl] = idx_v[f, sl] + off
        return 0

    lax.fori_loop(0, NUM_FIELDS, _off_body, 0)

    def _start_gather(k, slot_ref, sem):
        f = k // BLK_PER_W
        j = k % BLK_PER_W
        pltpu.make_async_copy(
            table_hbm.at[idx_v.at[f, pl.ds(j * ROWS_PER_OP, ROWS_PER_OP)]],
            slot_ref, sem).start()

    def _accumulate(k, slot_ref, sem):
        f = k // BLK_PER_W
        j = k % BLK_PER_W
        pltpu.make_async_copy(
            table_hbm.at[idx_v.at[f, pl.ds(j * ROWS_PER_OP, ROWS_PER_OP)]],
            slot_ref, sem).wait()

        @pl.when(f == 0)
        def _init():
            pltpu.sync_copy(slot_ref, acc_sh.at[ident_v.at[j]])

        @pl.when(f != 0)
        def _accum():
            pltpu.sync_copy(slot_ref, acc_sh.at[ident_v.at[j]], add=True)

    _start_gather(0, rows_v.at[0], sem0)

    def _pipe_body(kk, _):
        k0 = 2 * kk
        _start_gather(k0 + 1, rows_v.at[1], sem1)
        _accumulate(k0, rows_v.at[0], sem0)

        @pl.when(kk < OPS_PER_W // 2 - 1)
        def _next():
            _start_gather(k0 + 2, rows_v.at[0], sem0)

        _accumulate(k0 + 1, rows_v.at[1], sem1)
        return 0

    lax.fori_loop(0, OPS_PER_W // 2, _pipe_body, 0)

    pltpu.sync_copy(acc_sh.at[pl.ds(sid * ROWS_PER_W, ROWS_PER_W)], tmp_v)
    pltpu.sync_copy(tmp_v, out_hbm.at[pl.ds(wid * ROWS_PER_W, ROWS_PER_W)])


@jax.jit
def _run(idx_t, table):
    mesh = plsc.VectorSubcoreMesh(core_axis_name="c", subcore_axis_name="s")
    call = pl.kernel(
        _sc_body,
        mesh=mesh,
        compiler_params=pltpu.CompilerParams(use_tc_tiling_on_sc=False),
        out_type=jax.ShapeDtypeStruct((BATCH, NUM_CLASSES), jnp.float32),
        scratch_types=[
            pltpu.VMEM((NUM_FIELDS, ROWS_PER_W), jnp.int32),
            pltpu.VMEM((2, ROWS_PER_OP, NUM_CLASSES), jnp.float32),
            pltpu.VMEM((BLK_PER_W, ROWS_PER_OP), jnp.int32),
            pltpu.VMEM((ROWS_PER_W, NUM_CLASSES), jnp.float32),
            pltpu.VMEM_SHARED((16 * ROWS_PER_W, NUM_CLASSES), jnp.float32),
            pltpu.SemaphoreType.DMA,
            pltpu.SemaphoreType.DMA,
        ],
    )
    return call(idx_t, table)


def kernel(indices, table):
    idx_t = indices.astype(jnp.int32).T
    return _run(idx_t, table)
